# baseline (speedup 1.0000x reference)
import jax
import jax.numpy as jnp
from jax.experimental import pallas as pl
from jax.experimental.pallas import tpu as pltpu

N = 1_000_000
N_ACT = 16
BLK = 10_240
G = (N + BLK - 1) // BLK


def _pass1(oh_ref, strat_ref, belief_ref, out_ref, inv_ref, acc_ref):
    i = pl.program_id(0)
    s = strat_ref[...]
    oh = oh_ref[...]
    prob = jnp.sum(s * oh, axis=1)
    prod = prob * belief_ref[...]
    rows = i * BLK + jax.lax.iota(jnp.int32, BLK)
    prod = jnp.where(rows < N, prod, 0.0)
    out_ref[...] = prod

    @pl.when(i == 0)
    def _():
        acc_ref[0] = 0.0

    acc_ref[0] += jnp.sum(prod)

    @pl.when(i == pl.num_programs(0) - 1)
    def _():
        inv_ref[0, 0] = 1.0 / acc_ref[0]


def _pass2(prod_ref, inv_ref, out_ref):
    out_ref[...] = prod_ref[...] * inv_ref[0, 0]


def kernel(belief, strategy, action):
    onehot = (jax.lax.iota(jnp.int32, N_ACT) == action).astype(jnp.float32)
    onehot = onehot.reshape(1, N_ACT)
    prod, inv = pl.pallas_call(
        _pass1,
        grid=(G,),
        in_specs=[
            pl.BlockSpec((1, N_ACT), lambda i: (0, 0)),
            pl.BlockSpec((BLK, N_ACT), lambda i: (i, 0)),
            pl.BlockSpec((BLK,), lambda i: (i,)),
        ],
        out_specs=[
            pl.BlockSpec((BLK,), lambda i: (i,)),
            pl.BlockSpec(memory_space=pltpu.SMEM),
        ],
        out_shape=[
            jax.ShapeDtypeStruct((N,), jnp.float32),
            jax.ShapeDtypeStruct((1, 1), jnp.float32),
        ],
        scratch_shapes=[pltpu.SMEM((1,), jnp.float32)],
    )(onehot, strategy, belief)
    return pl.pallas_call(
        _pass2,
        grid=(G,),
        in_specs=[
            pl.BlockSpec((BLK,), lambda i: (i,)),
            pl.BlockSpec(memory_space=pltpu.SMEM),
        ],
        out_specs=pl.BlockSpec((BLK,), lambda i: (i,)),
        out_shape=jax.ShapeDtypeStruct((N,), jnp.float32),
    )(prod, inv)


# speedup vs baseline: 6.0522x; 6.0522x over previous
import jax
import jax.numpy as jnp
from jax.experimental import pallas as pl
from jax.experimental.pallas import tpu as pltpu

N = 1_000_000
N_ACT = 16
CB = 8_192
GF = N // CB
TAIL = N - GF * CB
G = GF + 1


def _chunk_copy(strat_hbm, colbuf, sems, a, j):
    slot = jax.lax.rem(j, 2)
    return pltpu.make_async_copy(
        strat_hbm.at[a, pl.ds(j * CB, CB)],
        colbuf.at[slot, 0],
        sems.at[slot],
    )


def _tail_copy(strat_hbm, tailbuf, sems, a):
    return pltpu.make_async_copy(
        strat_hbm.at[a, pl.ds(GF * CB, TAIL)],
        tailbuf.at[0],
        sems.at[2],
    )


def _body(act_ref, strat_hbm, belief_ref, out_ref,
          colbuf, tailbuf, prod_ref, acc_ref, sems):
    p = pl.program_id(0)
    i = pl.program_id(1)
    a = act_ref[0]

    @pl.when(p == 0)
    def _():
        @pl.when(i == 0)
        def _():
            acc_ref[0] = 0.0
            _chunk_copy(strat_hbm, colbuf, sems, a, 0).start()
            _tail_copy(strat_hbm, tailbuf, sems, a).start()

        @pl.when(i + 1 < GF)
        def _():
            _chunk_copy(strat_hbm, colbuf, sems, a, i + 1).start()

        @pl.when(i < GF)
        def _():
            _chunk_copy(strat_hbm, colbuf, sems, a, i).wait()
            slot = jax.lax.rem(i, 2)
            prod = colbuf[slot, 0, :] * belief_ref[...]
            prod_ref[pl.ds(i * CB, CB)] = prod
            acc_ref[0] += jnp.sum(prod)

        @pl.when(i == GF)
        def _():
            _tail_copy(strat_hbm, tailbuf, sems, a).wait()
            prod_t = tailbuf[0, :] * belief_ref[pl.ds(0, TAIL)]
            prod_ref[pl.ds(GF * CB, TAIL)] = prod_t
            acc_ref[0] += jnp.sum(prod_t)

    @pl.when(p == 1)
    def _():
        inv = 1.0 / acc_ref[0]

        @pl.when(i < GF)
        def _():
            out_ref[...] = prod_ref[pl.ds(i * CB, CB)] * inv

        @pl.when(i == GF)
        def _():
            out_ref[pl.ds(0, TAIL)] = prod_ref[pl.ds(GF * CB, TAIL)] * inv


def kernel(belief, strategy, action):
    strat_t = jnp.transpose(strategy)
    act = jnp.asarray(action, jnp.int32).reshape(1)
    grid_spec = pltpu.PrefetchScalarGridSpec(
        num_scalar_prefetch=1,
        grid=(2, G),
        in_specs=[
            pl.BlockSpec(memory_space=pltpu.HBM),
            pl.BlockSpec((CB,), lambda p, i, a: (jnp.where(p == 0, i, G - 1),)),
        ],
        out_specs=pl.BlockSpec((CB,), lambda p, i, a: (jnp.where(p == 0, 0, i),)),
        scratch_shapes=[
            pltpu.VMEM((2, 1, CB), jnp.float32),
            pltpu.VMEM((1, TAIL), jnp.float32),
            pltpu.VMEM((N,), jnp.float32),
            pltpu.SMEM((1,), jnp.float32),
            pltpu.SemaphoreType.DMA((3,)),
        ],
    )
    return pl.pallas_call(
        _body,
        grid_spec=grid_spec,
        out_shape=jax.ShapeDtypeStruct((N,), jnp.float32),
    )(act, strat_t, belief)


# speedup vs baseline: 31.9220x; 5.2745x over previous
import jax
import jax.numpy as jnp
from jax.experimental import pallas as pl
from jax.experimental.pallas import tpu as pltpu

N = 1_000_000
N_ACT = 16
CB = 65_536
GF = N // CB
TAIL = N - GF * CB
G = GF + 1


def _chunk_copy(strat_hbm, colbuf, sems, a, j):
    slot = jax.lax.rem(j, 2)
    return pltpu.make_async_copy(
        strat_hbm.at[a, pl.ds(j * CB, CB)],
        colbuf.at[slot, 0],
        sems.at[slot],
    )


def _tail_copy(strat_hbm, tailbuf, sems, a):
    return pltpu.make_async_copy(
        strat_hbm.at[a, pl.ds(GF * CB, TAIL)],
        tailbuf.at[0],
        sems.at[2],
    )


def _body(act_ref, strat_hbm, belief_ref, out_ref,
          colbuf, tailbuf, prod_ref, acc_ref, sems):
    p = pl.program_id(0)
    i = pl.program_id(1)
    a = act_ref[0]

    @pl.when(p == 0)
    def _():
        @pl.when(i == 0)
        def _():
            acc_ref[0] = 0.0
            _chunk_copy(strat_hbm, colbuf, sems, a, 0).start()
            _tail_copy(strat_hbm, tailbuf, sems, a).start()

        @pl.when(i + 1 < GF)
        def _():
            _chunk_copy(strat_hbm, colbuf, sems, a, i + 1).start()

        @pl.when(i < GF)
        def _():
            _chunk_copy(strat_hbm, colbuf, sems, a, i).wait()
            slot = jax.lax.rem(i, 2)
            prod = colbuf[slot, 0, :] * belief_ref[...]
            prod_ref[pl.ds(i * CB, CB)] = prod
            acc_ref[0] += jnp.sum(prod)

        @pl.when(i == GF)
        def _():
            _tail_copy(strat_hbm, tailbuf, sems, a).wait()
            prod_t = tailbuf[0, :] * belief_ref[pl.ds(0, TAIL)]
            prod_ref[pl.ds(GF * CB, TAIL)] = prod_t
            acc_ref[0] += jnp.sum(prod_t)

    @pl.when(p == 1)
    def _():
        inv = 1.0 / acc_ref[0]

        @pl.when(i < GF)
        def _():
            out_ref[...] = prod_ref[pl.ds(i * CB, CB)] * inv

        @pl.when(i == GF)
        def _():
            out_ref[pl.ds(0, TAIL)] = prod_ref[pl.ds(GF * CB, TAIL)] * inv


def kernel(belief, strategy, action):
    strat_t = jnp.transpose(strategy)
    act = jnp.asarray(action, jnp.int32).reshape(1)
    grid_spec = pltpu.PrefetchScalarGridSpec(
        num_scalar_prefetch=1,
        grid=(2, G),
        in_specs=[
            pl.BlockSpec(memory_space=pltpu.HBM),
            pl.BlockSpec((CB,), lambda p, i, a: (jnp.where(p == 0, i, G - 1),)),
        ],
        out_specs=pl.BlockSpec((CB,), lambda p, i, a: (jnp.where(p == 0, 0, i),)),
        scratch_shapes=[
            pltpu.VMEM((2, 1, CB), jnp.float32),
            pltpu.VMEM((1, TAIL), jnp.float32),
            pltpu.VMEM((N,), jnp.float32),
            pltpu.SMEM((1,), jnp.float32),
            pltpu.SemaphoreType.DMA((3,)),
        ],
    )
    return pl.pallas_call(
        _body,
        grid_spec=grid_spec,
        out_shape=jax.ShapeDtypeStruct((N,), jnp.float32),
    )(act, strat_t, belief)


# speedup vs baseline: 43.3201x; 1.3571x over previous
import jax
import jax.numpy as jnp
from jax.experimental import pallas as pl
from jax.experimental.pallas import tpu as pltpu

N = 1_000_000
N_ACT = 16
CB = 131_072
GF = N // CB
TAIL = N - GF * CB
G = GF + 1


def _chunk_copy(strat_hbm, colbuf, sems, a, j):
    slot = jax.lax.rem(j, 2)
    return pltpu.make_async_copy(
        strat_hbm.at[a, pl.ds(j * CB, CB)],
        colbuf.at[slot, 0],
        sems.at[slot],
    )


def _tail_copy(strat_hbm, tailbuf, sems, a):
    return pltpu.make_async_copy(
        strat_hbm.at[a, pl.ds(GF * CB, TAIL)],
        tailbuf.at[0],
        sems.at[2],
    )


def _body(act_ref, strat_hbm, belief_ref, out_ref,
          colbuf, tailbuf, prod_ref, acc_ref, sems):
    p = pl.program_id(0)
    i = pl.program_id(1)
    a = act_ref[0]

    @pl.when(p == 0)
    def _():
        @pl.when(i == 0)
        def _():
            acc_ref[0] = 0.0
            _chunk_copy(strat_hbm, colbuf, sems, a, 0).start()
            _tail_copy(strat_hbm, tailbuf, sems, a).start()

        @pl.when(i + 1 < GF)
        def _():
            _chunk_copy(strat_hbm, colbuf, sems, a, i + 1).start()

        @pl.when(i < GF)
        def _():
            _chunk_copy(strat_hbm, colbuf, sems, a, i).wait()
            slot = jax.lax.rem(i, 2)
            prod = colbuf[slot, 0, :] * belief_ref[...]
            prod_ref[pl.ds(i * CB, CB)] = prod
            acc_ref[0] += jnp.sum(prod)

        @pl.when(i == GF)
        def _():
            _tail_copy(strat_hbm, tailbuf, sems, a).wait()
            prod_t = tailbuf[0, :] * belief_ref[pl.ds(0, TAIL)]
            prod_ref[pl.ds(GF * CB, TAIL)] = prod_t
            acc_ref[0] += jnp.sum(prod_t)

    @pl.when(p == 1)
    def _():
        inv = 1.0 / acc_ref[0]

        @pl.when(i < GF)
        def _():
            out_ref[...] = prod_ref[pl.ds(i * CB, CB)] * inv

        @pl.when(i == GF)
        def _():
            out_ref[pl.ds(0, TAIL)] = prod_ref[pl.ds(GF * CB, TAIL)] * inv


def kernel(belief, strategy, action):
    strat_t = jnp.transpose(strategy)
    act = jnp.asarray(action, jnp.int32).reshape(1)
    grid_spec = pltpu.PrefetchScalarGridSpec(
        num_scalar_prefetch=1,
        grid=(2, G),
        in_specs=[
            pl.BlockSpec(memory_space=pltpu.HBM),
            pl.BlockSpec((CB,), lambda p, i, a: (jnp.where(p == 0, i, G - 1),)),
        ],
        out_specs=pl.BlockSpec((CB,), lambda p, i, a: (jnp.where(p == 0, 0, i),)),
        scratch_shapes=[
            pltpu.VMEM((2, 1, CB), jnp.float32),
            pltpu.VMEM((1, TAIL), jnp.float32),
            pltpu.VMEM((N,), jnp.float32),
            pltpu.SMEM((1,), jnp.float32),
            pltpu.SemaphoreType.DMA((3,)),
        ],
    )
    return pl.pallas_call(
        _body,
        grid_spec=grid_spec,
        out_shape=jax.ShapeDtypeStruct((N,), jnp.float32),
    )(act, strat_t, belief)


# speedup vs baseline: 45.2980x; 1.0457x over previous
import jax
import jax.numpy as jnp
from jax.experimental import pallas as pl
from jax.experimental.pallas import tpu as pltpu

N = 1_000_000
N_ACT = 16
CB = 131_072
GF = N // CB
TAIL = N - GF * CB
G = GF + 1


def _body(act_ref, strat_hbm, belief_hbm, out_ref,
          col_ref, ctail_ref, bel_ref, btail_ref, prod_ref, acc_ref,
          csems, bsems, tsems):
    p = pl.program_id(0)
    i = pl.program_id(1)
    a = act_ref[0]

    @pl.when(p == 0)
    def _():
        @pl.when(i == 0)
        def _():
            acc_ref[0] = 0.0
            for j in range(GF):
                pltpu.make_async_copy(
                    strat_hbm.at[a, pl.ds(j * CB, CB)],
                    col_ref.at[pl.ds(j * CB, CB)],
                    csems.at[j],
                ).start()
                pltpu.make_async_copy(
                    belief_hbm.at[pl.ds(j * CB, CB)],
                    bel_ref.at[pl.ds(j * CB, CB)],
                    bsems.at[j],
                ).start()
            pltpu.make_async_copy(
                strat_hbm.at[a, pl.ds(GF * CB, TAIL)], ctail_ref, tsems.at[0],
            ).start()
            pltpu.make_async_copy(
                belief_hbm.at[pl.ds(GF * CB, TAIL)], btail_ref, tsems.at[1],
            ).start()

        @pl.when(i < GF)
        def _():
            pltpu.make_async_copy(
                strat_hbm.at[a, pl.ds(0, CB)],
                col_ref.at[pl.ds(0, CB)],
                csems.at[i],
            ).wait()
            pltpu.make_async_copy(
                belief_hbm.at[pl.ds(0, CB)],
                bel_ref.at[pl.ds(0, CB)],
                bsems.at[i],
            ).wait()
            prod = col_ref[pl.ds(i * CB, CB)] * bel_ref[pl.ds(i * CB, CB)]
            prod_ref[pl.ds(i * CB, CB)] = prod
            acc_ref[0] += jnp.sum(prod)

        @pl.when(i == GF)
        def _():
            pltpu.make_async_copy(
                strat_hbm.at[a, pl.ds(GF * CB, TAIL)], ctail_ref, tsems.at[0],
            ).wait()
            pltpu.make_async_copy(
                belief_hbm.at[pl.ds(GF * CB, TAIL)], btail_ref, tsems.at[1],
            ).wait()
            prod_t = ctail_ref[...] * btail_ref[...]
            prod_ref[pl.ds(GF * CB, TAIL)] = prod_t
            acc_ref[0] += jnp.sum(prod_t)

    @pl.when(p == 1)
    def _():
        inv = 1.0 / acc_ref[0]

        @pl.when(i < GF)
        def _():
            out_ref[...] = prod_ref[pl.ds(i * CB, CB)] * inv

        @pl.when(i == GF)
        def _():
            out_ref[pl.ds(0, TAIL)] = prod_ref[pl.ds(GF * CB, TAIL)] * inv


def kernel(belief, strategy, action):
    strat_t = jnp.transpose(strategy)
    act = jnp.asarray(action, jnp.int32).reshape(1)
    grid_spec = pltpu.PrefetchScalarGridSpec(
        num_scalar_prefetch=1,
        grid=(2, G),
        in_specs=[
            pl.BlockSpec(memory_space=pltpu.HBM),
            pl.BlockSpec(memory_space=pltpu.HBM),
        ],
        out_specs=pl.BlockSpec((CB,), lambda p, i, a: (jnp.where(p == 0, 0, i),)),
        scratch_shapes=[
            pltpu.VMEM((GF * CB,), jnp.float32),
            pltpu.VMEM((TAIL,), jnp.float32),
            pltpu.VMEM((GF * CB,), jnp.float32),
            pltpu.VMEM((TAIL,), jnp.float32),
            pltpu.VMEM((N,), jnp.float32),
            pltpu.SMEM((1,), jnp.float32),
            pltpu.SemaphoreType.DMA((GF,)),
            pltpu.SemaphoreType.DMA((GF,)),
            pltpu.SemaphoreType.DMA((2,)),
        ],
    )
    return pl.pallas_call(
        _body,
        grid_spec=grid_spec,
        out_shape=jax.ShapeDtypeStruct((N,), jnp.float32),
    )(act, strat_t, belief)
